# baseline (speedup 1.0000x reference)
import functools

import jax
import jax.numpy as jnp
from jax import lax
from jax.experimental import pallas as pl
from jax.experimental.pallas import tpu as pltpu
from jax.experimental.pallas import tpu_sc as plsc

N = 100000
E = 1600000
L = 3
D = 32

NT = 100096
E_PAD = 1605632
ER = E_PAD // 128
EPW = E_PAD // 16
CHUNKS = 392
ZROWS = NT // 16

_BN = 1024
_BE = 6272


def _pack_wide(x, n, w=16):
    g = 128 // w
    h3 = jnp.reshape(x, (n // g, g, w))
    return jnp.concatenate([h3[:, k, :] for k in range(g)], axis=1)


def _unpack_wide(wide, bn, w=16):
    g = 128 // w
    parts = [wide[:, k * w:(k + 1) * w] for k in range(g)]
    st = jnp.concatenate([p[:, None, :] for p in parts], axis=1)
    return jnp.reshape(st, (bn, w))


def _node_mlp_body(x_ref, W1a_ref, b1a_ref, W1b_ref, b1b_ref, out_ref):
    c = pl.program_id(0)
    t = jnp.maximum(
        jnp.dot(x_ref[...], W1a_ref[...], preferred_element_type=jnp.float32)
        + b1a_ref[...], 0.0)
    val = (jnp.dot(t, W1b_ref[...], preferred_element_type=jnp.float32)
           + b1b_ref[...])
    half = jnp.where(c == 0, val[:, :16], val[:, 16:])
    out_ref[...] = half[None]


def _node_mlp(x, W1a, b1a, W1b, b1b):
    return pl.pallas_call(
        _node_mlp_body,
        grid=(2, pl.cdiv(N, _BN)),
        in_specs=[
            pl.BlockSpec((_BN, 12), lambda c, i: (i, 0)),
            pl.BlockSpec((12, 27), lambda c, i: (0, 0)),
            pl.BlockSpec((1, 27), lambda c, i: (0, 0)),
            pl.BlockSpec((27, 32), lambda c, i: (0, 0)),
            pl.BlockSpec((1, 32), lambda c, i: (0, 0)),
        ],
        out_specs=pl.BlockSpec((1, _BN, 16), lambda c, i: (c, i, 0)),
        out_shape=jax.ShapeDtypeStruct((2, NT, 16), jnp.float32),
    )(x, W1a, b1a.reshape(1, 27), W1b, b1b.reshape(1, 32))


EW_ROWS = E_PAD // 32
_BEW = 512


def _edge_mlp_body(ew_ref, WA_ref, bA_ref, WB0_ref, bB0_ref, WB1_ref,
                   bB1_ref, o0_ref, o1_ref, o2_ref, o3_ref):
    xw = ew_ref[...]
    outs = (o0_ref, o1_ref, o2_ref, o3_ref)
    for q in range(4):
        x8 = xw[:, 32 * q:32 * q + 32]
        t8 = jnp.maximum(
            jnp.dot(x8, WA_ref[...], preferred_element_type=jnp.float32)
            + bA_ref[...], 0.0)
        h0 = (jnp.dot(t8, WB0_ref[...], preferred_element_type=jnp.float32)
              + bB0_ref[...])
        h1 = (jnp.dot(t8, WB1_ref[...], preferred_element_type=jnp.float32)
              + bB1_ref[...])
        outs[q][...] = jnp.concatenate([h0[None], h1[None]], axis=0)


def _edge_mlp(edge_attr, W2a, b2a, W2b, b2b):
    ew = jnp.reshape(jnp.pad(edge_attr, ((0, 0), (0, 1))), (E * 4 // 128, 128))
    eye8 = jnp.eye(8, dtype=jnp.float32)
    W2a4 = jnp.pad(W2a, ((0, 1), (0, 0)))
    WA = jnp.kron(eye8, W2a4)
    bA = jnp.tile(b2a, 8).reshape(1, 72)
    WB0 = jnp.kron(eye8, W2b[:, :16])
    WB1 = jnp.kron(eye8, W2b[:, 16:])
    bB0 = jnp.tile(b2b[:16], 8).reshape(1, 128)
    bB1 = jnp.tile(b2b[16:], 8).reshape(1, 128)
    full = lambda r, c_: pl.BlockSpec((r, c_), lambda i: (0, 0))
    ospec = pl.BlockSpec((2, _BEW, 128), lambda i: (0, i, 0))
    oshape = jax.ShapeDtypeStruct((2, EW_ROWS, 128), jnp.float32)
    outs = pl.pallas_call(
        _edge_mlp_body,
        grid=(pl.cdiv(EW_ROWS, _BEW),),
        in_specs=[
            pl.BlockSpec((_BEW, 128), lambda i: (i, 0)),
            full(32, 72), full(1, 72),
            full(72, 128), full(1, 128), full(72, 128), full(1, 128),
        ],
        out_specs=[ospec] * 4,
        out_shape=[oshape] * 4,
    )(ew, WA, bA, WB0, bB0, WB1, bB1)
    return tuple(o.reshape(2 * EW_ROWS, 128) for o in outs)


CHUNK = 256
RPW = EPW // 128


def _sc_aggr_body(h2_hbm, ea0_hbm, ea1_hbm, ea2_hbm, ea3_hbm,
                  src_hbm, dst_hbm, out_hbm,
                  aggr_sh, sidx, didx, rows, eab,
                  lsem0, lsem1, dsem0, dsem1, gsem0, gsem1, ssem0, ssem1):
    c = lax.axis_index("c")
    s = lax.axis_index("s")
    lsem = (lsem0, lsem1)
    dsem = (dsem0, dsem1)
    gsem = (gsem0, gsem1)
    ssem = (ssem0, ssem1)

    @plsc.parallel_loop(0, 512)
    def _z(i):
        rows[i] = jnp.zeros((16,), jnp.float32)
    for k in range(12):
        pltpu.sync_copy(rows, aggr_sh.at[pl.ds(s * ZROWS + k * 512, 512)])
    pltpu.sync_copy(rows.at[pl.ds(0, 112)],
                    aggr_sh.at[pl.ds(s * ZROWS + 6144, 112)])
    plsc.subcore_barrier()

    ea_hbm = (ea0_hbm, ea1_hbm, ea2_hbm, ea3_hbm)

    def fire_loads(g, b):
        rbase = s * RPW + g * 2
        ebase = c * EW_ROWS + s * (EPW // 32) + g * 8
        pltpu.async_copy(src_hbm.at[pl.ds(c * ER + rbase, 2)],
                         sidx.at[pl.ds(b * 2, 2)], lsem[b])
        for q in range(4):
            pltpu.async_copy(ea_hbm[q].at[pl.ds(ebase, 8)],
                             eab.at[pl.ds(b * 32 + q * 8, 8)], lsem[b])

    def fire_didx(g, b):
        rbase = s * RPW + g * 2
        pltpu.async_copy(dst_hbm.at[pl.ds(rbase, 2)],
                         didx.at[pl.ds(b * 2, 2)], dsem[b])

    def wait_loads(b):
        pltpu.make_async_copy(src_hbm.at[pl.ds(0, 2)],
                              sidx.at[pl.ds(b * 2, 2)], lsem[b]).wait()
        for q in range(4):
            pltpu.make_async_copy(ea_hbm[q].at[pl.ds(0, 8)],
                                  eab.at[pl.ds(b * 32 + q * 8, 8)],
                                  lsem[b]).wait()

    def wait_didx(b):
        pltpu.make_async_copy(dst_hbm.at[pl.ds(0, 2)],
                              didx.at[pl.ds(b * 2, 2)], dsem[b]).wait()

    def fire_gathers(b):
        for j in range(2):
            pltpu.async_copy(h2_hbm.at[sidx.at[b * 2 + j]],
                             rows.at[pl.ds((b * 2 + j) * 128, 128)], gsem[b])

    def wait_gathers(b):
        for j in range(2):
            pltpu.make_async_copy(
                h2_hbm.at[pl.ds(0, 128)],
                rows.at[pl.ds((b * 2 + j) * 128, 128)], gsem[b]).wait()

    def fire_scatters(b):
        for j in range(2):
            pltpu.async_copy(rows.at[pl.ds((b * 2 + j) * 128, 128)],
                             aggr_sh.at[didx.at[b * 2 + j]], ssem[b], add=True)

    def wait_scatters(b):
        for j in range(2):
            pltpu.make_async_copy(rows.at[pl.ds((b * 2 + j) * 128, 128)],
                                  aggr_sh.at[pl.ds(0, 128)], ssem[b]).wait()

    def compute(b):
        @plsc.parallel_loop(0, 8, unroll=2)
        def _cmp(r):
            for q in range(4):
                for j in range(8):
                    k = b * CHUNK + 32 * r + 8 * q + j
                    e = eab[b * 32 + q * 8 + r, pl.ds(j * 16, 16)]
                    rows[k] = jnp.maximum(rows[k] + e, 0.0)

    fire_loads(0, 0)
    fire_didx(0, 0)
    wait_loads(0)
    fire_gathers(0)
    fire_loads(1, 1)

    def step(g, b, first, fire_next_gather, fire_next_loads):
        if fire_next_gather:
            wait_loads(1 - b)
        if first:
            wait_scatters(1 - b)
        if fire_next_gather:
            fire_didx(g + 1, 1 - b)
            fire_gathers(1 - b)
        wait_gathers(b)
        compute(b)
        wait_didx(b)
        fire_scatters(b)
        if fire_next_loads:
            fire_loads(g + 2, b)

    def body(gg, carry):
        g0 = gg * 2

        @pl.when(gg > 0)
        def _():
            wait_scatters(1)
        wait_loads(1)
        fire_didx(g0 + 1, 1)
        fire_gathers(1)
        wait_gathers(0)
        compute(0)
        wait_didx(0)
        fire_scatters(0)
        fire_loads(g0 + 2, 0)

        step(g0 + 1, 1, True, True, True)
        return carry

    lax.fori_loop(0, CHUNKS // 2 - 1, body, 0)
    step(CHUNKS - 2, 0, True, True, False)
    step(CHUNKS - 1, 1, True, False, False)
    wait_scatters(1)

    plsc.subcore_barrier()
    for k in range(12):
        off = s * ZROWS + k * 512
        pltpu.sync_copy(aggr_sh.at[pl.ds(off, 512)],
                        out_hbm.at[pl.ds(c * NT + off, 512)])
    off = s * ZROWS + 6144
    pltpu.sync_copy(aggr_sh.at[pl.ds(off, 112)],
                    out_hbm.at[pl.ds(c * NT + off, 112)])


def _sc_aggregate(h2_flat, eaq, src2, dstp):
    mesh = plsc.VectorSubcoreMesh(core_axis_name="c", subcore_axis_name="s")
    f = pl.kernel(
        _sc_aggr_body,
        out_type=jax.ShapeDtypeStruct((2 * NT, 16), jnp.float32),
        mesh=mesh,
        compiler_params=pltpu.CompilerParams(use_tc_tiling_on_sc=False),
        scratch_types=[
            pltpu.VMEM_SHARED((NT, 16), jnp.float32),
            pltpu.VMEM((4, 128), jnp.int32),
            pltpu.VMEM((4, 128), jnp.int32),
            pltpu.VMEM((512, 16), jnp.float32),
            pltpu.VMEM((64, 128), jnp.float32),
        ] + [pltpu.SemaphoreType.DMA] * 8,
    )
    return f(h2_flat, eaq[0], eaq[1], eaq[2], eaq[3], src2, dstp)


def _layer_mlp_body(h0_ref, h1_ref, a0_ref, a1_ref, Wc1_ref, bc1_ref,
                    Wc2_ref, bc2_ref, z_ref, sums_ref):
    i = pl.program_id(0)
    z0 = jnp.concatenate(
        [h0_ref[...] + a0_ref[...], h1_ref[...] + a1_ref[...]], axis=1)
    t = jnp.maximum(
        jnp.dot(z0, Wc1_ref[...], preferred_element_type=jnp.float32)
        + bc1_ref[...], 0.0)
    z = jnp.dot(t, Wc2_ref[...], preferred_element_type=jnp.float32) + bc2_ref[...]
    z_ref[...] = z
    row = lax.broadcasted_iota(jnp.int32, (_BN, 1), 0) + i * _BN
    zm = jnp.where(row < N, z, 0.0)
    blk = jnp.concatenate(
        [jnp.sum(zm, axis=0, keepdims=True),
         jnp.sum(zm * zm, axis=0, keepdims=True)], axis=0)

    @pl.when(i == 0)
    def _():
        sums_ref[...] = blk

    @pl.when(i != 0)
    def _():
        sums_ref[...] += blk


def _layer_mlp(h2, aggr, Wc1, bc1, Wc2, bc2):
    half = pl.BlockSpec((_BN, 16), lambda i: (i, 0))
    return pl.pallas_call(
        _layer_mlp_body,
        grid=(pl.cdiv(N, _BN),),
        in_specs=[
            half, half, half, half,
            pl.BlockSpec((32, 64), lambda i: (0, 0)),
            pl.BlockSpec((1, 64), lambda i: (0, 0)),
            pl.BlockSpec((64, 32), lambda i: (0, 0)),
            pl.BlockSpec((1, 32), lambda i: (0, 0)),
        ],
        out_specs=[
            pl.BlockSpec((_BN, 32), lambda i: (i, 0)),
            pl.BlockSpec((2, 32), lambda i: (0, 0)),
        ],
        out_shape=[
            jax.ShapeDtypeStruct((N, 32), jnp.float32),
            jax.ShapeDtypeStruct((2, 32), jnp.float32),
        ],
    )(h2[:N], h2[NT:NT + N], aggr[:N], aggr[NT:NT + N],
      Wc1, bc1.reshape(1, 64), Wc2, bc2.reshape(1, 32))


def _bn_mid_body(z_ref, sc_ref, sh_ref, out_ref):
    c = pl.program_id(0)
    val = jnp.maximum(z_ref[...] * sc_ref[...] + sh_ref[...], 0.0)
    half = jnp.where(c == 0, val[:, :16], val[:, 16:])
    out_ref[...] = half[None]


def _bn_mid(z, scale, shift):
    return pl.pallas_call(
        _bn_mid_body,
        grid=(2, pl.cdiv(N, _BN)),
        in_specs=[
            pl.BlockSpec((_BN, 32), lambda c, i: (i, 0)),
            pl.BlockSpec((1, 32), lambda c, i: (0, 0)),
            pl.BlockSpec((1, 32), lambda c, i: (0, 0)),
        ],
        out_specs=pl.BlockSpec((1, _BN, 16), lambda c, i: (c, i, 0)),
        out_shape=jax.ShapeDtypeStruct((2, NT, 16), jnp.float32),
    )(z, scale, shift)


def _bn_final_body(z_ref, sc_ref, sh_ref, out_ref):
    out_ref[...] = z_ref[...] * sc_ref[...] + sh_ref[...]


def _bn_final(z, scale, shift):
    return pl.pallas_call(
        _bn_final_body,
        grid=(pl.cdiv(N, _BN),),
        in_specs=[
            pl.BlockSpec((_BN, 32), lambda i: (i, 0)),
            pl.BlockSpec((1, 32), lambda i: (0, 0)),
            pl.BlockSpec((1, 32), lambda i: (0, 0)),
        ],
        out_specs=pl.BlockSpec((_BN, 32), lambda i: (i, 0)),
        out_shape=jax.ShapeDtypeStruct((N, 32), jnp.float32),
    )(z, scale, shift)


def kernel(x, edge_attr, edge_index, W1a, b1a, W1b, b1b, W2a, b2a, W2b, b2b,
           Wc1, bc1, Wc2, bc2, gamma, beta):
    src = edge_index[0]
    dst = edge_index[1]

    pad = N + (jnp.arange(E_PAD - E, dtype=jnp.int32) % 16)
    srcp = jnp.concatenate([src, pad])
    dstp = jnp.concatenate([dst, pad]).reshape(ER, 128)
    src2 = (srcp[None, :]
            + jnp.array([0, NT], dtype=jnp.int32)[:, None]).reshape(2 * ER, 128)

    h2 = _node_mlp(x, W1a, b1a, W1b, b1b).reshape(2 * NT, 16)
    eaq = _edge_mlp(edge_attr, W2a, b2a, W2b, b2b)

    for l in range(L):
        aggr = _sc_aggregate(h2, eaq, src2, dstp)
        z, sums = _layer_mlp(h2, aggr, Wc1[l], bc1[l], Wc2[l], bc2[l])
        mean = sums[0] / N
        var = sums[1] / N - mean * mean
        scale = (gamma[l] / jnp.sqrt(var + 1e-5)).reshape(1, 32)
        shift = (beta[l] - mean * scale[0]).reshape(1, 32)
        if l != L - 1:
            h2 = _bn_mid(z, scale, shift).reshape(2 * NT, 16)
        else:
            out = _bn_final(z, scale, shift)
    return out


# speedup vs baseline: 2.1045x; 2.1045x over previous
import functools

import jax
import jax.numpy as jnp
from jax import lax
from jax.experimental import pallas as pl
from jax.experimental.pallas import tpu as pltpu
from jax.experimental.pallas import tpu_sc as plsc

N = 100000
E = 1600000
L = 3
D = 32

NT = 100096
E_PAD = 1605632
ER = E_PAD // 128
EPW = E_PAD // 16
CHUNKS = 392
ZROWS = NT // 16

_BN = 1024
_BE = 6272


def _pack_wide(x, n, w=16):
    g = 128 // w
    h3 = jnp.reshape(x, (n // g, g, w))
    return jnp.concatenate([h3[:, k, :] for k in range(g)], axis=1)


def _unpack_wide(wide, bn, w=16):
    g = 128 // w
    parts = [wide[:, k * w:(k + 1) * w] for k in range(g)]
    st = jnp.concatenate([p[:, None, :] for p in parts], axis=1)
    return jnp.reshape(st, (bn, w))


def _node_mlp_body(x_ref, W1a_ref, b1a_ref, W1b_ref, b1b_ref, out_ref):
    c = pl.program_id(0)
    t = jnp.maximum(
        jnp.dot(x_ref[...], W1a_ref[...], preferred_element_type=jnp.float32)
        + b1a_ref[...], 0.0)
    val = (jnp.dot(t, W1b_ref[...], preferred_element_type=jnp.float32)
           + b1b_ref[...])
    half = jnp.where(c == 0, val[:, :16], val[:, 16:])
    out_ref[...] = half[None]


def _node_mlp(x, W1a, b1a, W1b, b1b):
    return pl.pallas_call(
        _node_mlp_body,
        grid=(2, pl.cdiv(N, _BN)),
        in_specs=[
            pl.BlockSpec((_BN, 12), lambda c, i: (i, 0)),
            pl.BlockSpec((12, 27), lambda c, i: (0, 0)),
            pl.BlockSpec((1, 27), lambda c, i: (0, 0)),
            pl.BlockSpec((27, 32), lambda c, i: (0, 0)),
            pl.BlockSpec((1, 32), lambda c, i: (0, 0)),
        ],
        out_specs=pl.BlockSpec((1, _BN, 16), lambda c, i: (c, i, 0)),
        out_shape=jax.ShapeDtypeStruct((2, NT, 16), jnp.float32),
    )(x, W1a, b1a.reshape(1, 27), W1b, b1b.reshape(1, 32))


EW_ROWS = E_PAD // 32
_BEW = 512
_BE3 = 32 * _BEW


def _ew_pack_body(ea_ref, out_ref):
    x = ea_ref[...]
    x4 = jnp.concatenate([x, jnp.zeros((_BE3, 1), jnp.float32)], axis=1)
    out_ref[...] = _pack_wide(x4, _BE3, 4)


def _ew_pack(edge_attr):
    return pl.pallas_call(
        _ew_pack_body,
        grid=(pl.cdiv(EW_ROWS, _BEW),),
        in_specs=[pl.BlockSpec((_BE3, 3), lambda i: (i, 0))],
        out_specs=pl.BlockSpec((_BEW, 128), lambda i: (i, 0)),
        out_shape=jax.ShapeDtypeStruct((EW_ROWS, 128), jnp.float32),
    )(edge_attr)


def _edge_mlp_body(ew_ref, WA_ref, bA_ref, WB0_ref, bB0_ref, WB1_ref,
                   bB1_ref, o0_ref, o1_ref, o2_ref, o3_ref):
    xw = ew_ref[...]
    outs = (o0_ref, o1_ref, o2_ref, o3_ref)
    for q in range(4):
        x8 = xw[:, 32 * q:32 * q + 32]
        t8 = jnp.maximum(
            jnp.dot(x8, WA_ref[...], preferred_element_type=jnp.float32)
            + bA_ref[...], 0.0)
        h0 = (jnp.dot(t8, WB0_ref[...], preferred_element_type=jnp.float32)
              + bB0_ref[...])
        h1 = (jnp.dot(t8, WB1_ref[...], preferred_element_type=jnp.float32)
              + bB1_ref[...])
        outs[q][...] = jnp.concatenate([h0[None], h1[None]], axis=0)


def _edge_mlp(edge_attr, W2a, b2a, W2b, b2b):
    ew = _ew_pack(edge_attr)
    eye8 = jnp.eye(8, dtype=jnp.float32)
    W2a4 = jnp.pad(W2a, ((0, 1), (0, 0)))
    WA = jnp.kron(eye8, W2a4)
    bA = jnp.tile(b2a, 8).reshape(1, 72)
    WB0 = jnp.kron(eye8, W2b[:, :16])
    WB1 = jnp.kron(eye8, W2b[:, 16:])
    bB0 = jnp.tile(b2b[:16], 8).reshape(1, 128)
    bB1 = jnp.tile(b2b[16:], 8).reshape(1, 128)
    full = lambda r, c_: pl.BlockSpec((r, c_), lambda i: (0, 0))
    ospec = pl.BlockSpec((2, _BEW, 128), lambda i: (0, i, 0))
    oshape = jax.ShapeDtypeStruct((2, EW_ROWS, 128), jnp.float32)
    outs = pl.pallas_call(
        _edge_mlp_body,
        grid=(pl.cdiv(EW_ROWS, _BEW),),
        in_specs=[
            pl.BlockSpec((_BEW, 128), lambda i: (i, 0)),
            full(32, 72), full(1, 72),
            full(72, 128), full(1, 128), full(72, 128), full(1, 128),
        ],
        out_specs=[ospec] * 4,
        out_shape=[oshape] * 4,
    )(ew, WA, bA, WB0, bB0, WB1, bB1)
    return tuple(o.reshape(2 * EW_ROWS, 128) for o in outs)


CHUNK = 256
RPW = EPW // 128


def _sc_aggr_body(h2_hbm, ea0_hbm, ea1_hbm, ea2_hbm, ea3_hbm,
                  src_hbm, dst_hbm, out_hbm,
                  aggr_sh, sidx, didx, rows, eab,
                  lsem0, lsem1, dsem0, dsem1, gsem0, gsem1, ssem0, ssem1):
    c = lax.axis_index("c")
    s = lax.axis_index("s")
    lsem = (lsem0, lsem1)
    dsem = (dsem0, dsem1)
    gsem = (gsem0, gsem1)
    ssem = (ssem0, ssem1)

    @plsc.parallel_loop(0, 512)
    def _z(i):
        rows[i] = jnp.zeros((16,), jnp.float32)
    for k in range(12):
        pltpu.sync_copy(rows, aggr_sh.at[pl.ds(s * ZROWS + k * 512, 512)])
    pltpu.sync_copy(rows.at[pl.ds(0, 112)],
                    aggr_sh.at[pl.ds(s * ZROWS + 6144, 112)])
    plsc.subcore_barrier()

    ea_hbm = (ea0_hbm, ea1_hbm, ea2_hbm, ea3_hbm)

    def fire_loads(g, b):
        rbase = s * RPW + g * 2
        ebase = c * EW_ROWS + s * (EPW // 32) + g * 8
        pltpu.async_copy(src_hbm.at[pl.ds(c * ER + rbase, 2)],
                         sidx.at[pl.ds(b * 2, 2)], lsem[b])
        for q in range(4):
            pltpu.async_copy(ea_hbm[q].at[pl.ds(ebase, 8)],
                             eab.at[pl.ds(b * 32 + q * 8, 8)], lsem[b])

    def fire_didx(g, b):
        rbase = s * RPW + g * 2
        pltpu.async_copy(dst_hbm.at[pl.ds(rbase, 2)],
                         didx.at[pl.ds(b * 2, 2)], dsem[b])

    def wait_loads(b):
        pltpu.make_async_copy(src_hbm.at[pl.ds(0, 2)],
                              sidx.at[pl.ds(b * 2, 2)], lsem[b]).wait()
        for q in range(4):
            pltpu.make_async_copy(ea_hbm[q].at[pl.ds(0, 8)],
                                  eab.at[pl.ds(b * 32 + q * 8, 8)],
                                  lsem[b]).wait()

    def wait_didx(b):
        pltpu.make_async_copy(dst_hbm.at[pl.ds(0, 2)],
                              didx.at[pl.ds(b * 2, 2)], dsem[b]).wait()

    def fire_gathers(b):
        for j in range(2):
            pltpu.async_copy(h2_hbm.at[sidx.at[b * 2 + j]],
                             rows.at[pl.ds((b * 2 + j) * 128, 128)], gsem[b])

    def wait_gathers(b):
        for j in range(2):
            pltpu.make_async_copy(
                h2_hbm.at[pl.ds(0, 128)],
                rows.at[pl.ds((b * 2 + j) * 128, 128)], gsem[b]).wait()

    def fire_scatters(b):
        for j in range(2):
            pltpu.async_copy(rows.at[pl.ds((b * 2 + j) * 128, 128)],
                             aggr_sh.at[didx.at[b * 2 + j]], ssem[b], add=True)

    def wait_scatters(b):
        for j in range(2):
            pltpu.make_async_copy(rows.at[pl.ds((b * 2 + j) * 128, 128)],
                                  aggr_sh.at[pl.ds(0, 128)], ssem[b]).wait()

    def compute(b):
        @plsc.parallel_loop(0, 8, unroll=2)
        def _cmp(r):
            for q in range(4):
                for j in range(8):
                    k = b * CHUNK + 32 * r + 8 * q + j
                    e = eab[b * 32 + q * 8 + r, pl.ds(j * 16, 16)]
                    rows[k] = jnp.maximum(rows[k] + e, 0.0)

    fire_loads(0, 0)
    fire_didx(0, 0)
    wait_loads(0)
    fire_gathers(0)
    fire_loads(1, 1)

    def step(g, b, first, fire_next_gather, fire_next_loads):
        if fire_next_gather:
            wait_loads(1 - b)
        if first:
            wait_scatters(1 - b)
        if fire_next_gather:
            fire_didx(g + 1, 1 - b)
            fire_gathers(1 - b)
        wait_gathers(b)
        compute(b)
        wait_didx(b)
        fire_scatters(b)
        if fire_next_loads:
            fire_loads(g + 2, b)

    def body(gg, carry):
        g0 = gg * 2

        @pl.when(gg > 0)
        def _():
            wait_scatters(1)
        wait_loads(1)
        fire_didx(g0 + 1, 1)
        fire_gathers(1)
        wait_gathers(0)
        compute(0)
        wait_didx(0)
        fire_scatters(0)
        fire_loads(g0 + 2, 0)

        step(g0 + 1, 1, True, True, True)
        return carry

    lax.fori_loop(0, CHUNKS // 2 - 1, body, 0)
    step(CHUNKS - 2, 0, True, True, False)
    step(CHUNKS - 1, 1, True, False, False)
    wait_scatters(1)

    plsc.subcore_barrier()
    for k in range(12):
        off = s * ZROWS + k * 512
        pltpu.sync_copy(aggr_sh.at[pl.ds(off, 512)],
                        out_hbm.at[pl.ds(c * NT + off, 512)])
    off = s * ZROWS + 6144
    pltpu.sync_copy(aggr_sh.at[pl.ds(off, 112)],
                    out_hbm.at[pl.ds(c * NT + off, 112)])


def _sc_aggregate(h2_flat, eaq, src2, dstp):
    mesh = plsc.VectorSubcoreMesh(core_axis_name="c", subcore_axis_name="s")
    f = pl.kernel(
        _sc_aggr_body,
        out_type=jax.ShapeDtypeStruct((2 * NT, 16), jnp.float32),
        mesh=mesh,
        compiler_params=pltpu.CompilerParams(use_tc_tiling_on_sc=False),
        scratch_types=[
            pltpu.VMEM_SHARED((NT, 16), jnp.float32),
            pltpu.VMEM((4, 128), jnp.int32),
            pltpu.VMEM((4, 128), jnp.int32),
            pltpu.VMEM((512, 16), jnp.float32),
            pltpu.VMEM((64, 128), jnp.float32),
        ] + [pltpu.SemaphoreType.DMA] * 8,
    )
    return f(h2_flat, eaq[0], eaq[1], eaq[2], eaq[3], src2, dstp)


def _layer_mlp_body(h0_ref, h1_ref, a0_ref, a1_ref, Wc1_ref, bc1_ref,
                    Wc2_ref, bc2_ref, z_ref, sums_ref):
    i = pl.program_id(0)
    z0 = jnp.concatenate(
        [h0_ref[...] + a0_ref[...], h1_ref[...] + a1_ref[...]], axis=1)
    t = jnp.maximum(
        jnp.dot(z0, Wc1_ref[...], preferred_element_type=jnp.float32)
        + bc1_ref[...], 0.0)
    z = jnp.dot(t, Wc2_ref[...], preferred_element_type=jnp.float32) + bc2_ref[...]
    z_ref[...] = z
    row = lax.broadcasted_iota(jnp.int32, (_BN, 1), 0) + i * _BN
    zm = jnp.where(row < N, z, 0.0)
    blk = jnp.concatenate(
        [jnp.sum(zm, axis=0, keepdims=True),
         jnp.sum(zm * zm, axis=0, keepdims=True)], axis=0)

    @pl.when(i == 0)
    def _():
        sums_ref[...] = blk

    @pl.when(i != 0)
    def _():
        sums_ref[...] += blk


def _layer_mlp(h2, aggr, Wc1, bc1, Wc2, bc2):
    half = pl.BlockSpec((_BN, 16), lambda i: (i, 0))
    return pl.pallas_call(
        _layer_mlp_body,
        grid=(pl.cdiv(N, _BN),),
        in_specs=[
            half, half, half, half,
            pl.BlockSpec((32, 64), lambda i: (0, 0)),
            pl.BlockSpec((1, 64), lambda i: (0, 0)),
            pl.BlockSpec((64, 32), lambda i: (0, 0)),
            pl.BlockSpec((1, 32), lambda i: (0, 0)),
        ],
        out_specs=[
            pl.BlockSpec((_BN, 32), lambda i: (i, 0)),
            pl.BlockSpec((2, 32), lambda i: (0, 0)),
        ],
        out_shape=[
            jax.ShapeDtypeStruct((N, 32), jnp.float32),
            jax.ShapeDtypeStruct((2, 32), jnp.float32),
        ],
    )(h2[:N], h2[NT:NT + N], aggr[:N], aggr[NT:NT + N],
      Wc1, bc1.reshape(1, 64), Wc2, bc2.reshape(1, 32))


def _bn_mid_body(z_ref, sc_ref, sh_ref, out_ref):
    c = pl.program_id(0)
    val = jnp.maximum(z_ref[...] * sc_ref[...] + sh_ref[...], 0.0)
    half = jnp.where(c == 0, val[:, :16], val[:, 16:])
    out_ref[...] = half[None]


def _bn_mid(z, scale, shift):
    return pl.pallas_call(
        _bn_mid_body,
        grid=(2, pl.cdiv(N, _BN)),
        in_specs=[
            pl.BlockSpec((_BN, 32), lambda c, i: (i, 0)),
            pl.BlockSpec((1, 32), lambda c, i: (0, 0)),
            pl.BlockSpec((1, 32), lambda c, i: (0, 0)),
        ],
        out_specs=pl.BlockSpec((1, _BN, 16), lambda c, i: (c, i, 0)),
        out_shape=jax.ShapeDtypeStruct((2, NT, 16), jnp.float32),
    )(z, scale, shift)


def _bn_final_body(z_ref, sc_ref, sh_ref, out_ref):
    out_ref[...] = z_ref[...] * sc_ref[...] + sh_ref[...]


def _bn_final(z, scale, shift):
    return pl.pallas_call(
        _bn_final_body,
        grid=(pl.cdiv(N, _BN),),
        in_specs=[
            pl.BlockSpec((_BN, 32), lambda i: (i, 0)),
            pl.BlockSpec((1, 32), lambda i: (0, 0)),
            pl.BlockSpec((1, 32), lambda i: (0, 0)),
        ],
        out_specs=pl.BlockSpec((_BN, 32), lambda i: (i, 0)),
        out_shape=jax.ShapeDtypeStruct((N, 32), jnp.float32),
    )(z, scale, shift)


def kernel(x, edge_attr, edge_index, W1a, b1a, W1b, b1b, W2a, b2a, W2b, b2b,
           Wc1, bc1, Wc2, bc2, gamma, beta):
    src = edge_index[0]
    dst = edge_index[1]

    pad = N + (jnp.arange(E_PAD - E, dtype=jnp.int32) % 16)
    srcp = jnp.concatenate([src, pad])
    dstp = jnp.concatenate([dst, pad]).reshape(ER, 128)
    src2 = (srcp[None, :]
            + jnp.array([0, NT], dtype=jnp.int32)[:, None]).reshape(2 * ER, 128)

    h2 = _node_mlp(x, W1a, b1a, W1b, b1b).reshape(2 * NT, 16)
    eaq = _edge_mlp(edge_attr, W2a, b2a, W2b, b2b)

    for l in range(L):
        aggr = _sc_aggregate(h2, eaq, src2, dstp)
        z, sums = _layer_mlp(h2, aggr, Wc1[l], bc1[l], Wc2[l], bc2[l])
        mean = sums[0] / N
        var = sums[1] / N - mean * mean
        scale = (gamma[l] / jnp.sqrt(var + 1e-5)).reshape(1, 32)
        shift = (beta[l] - mean * scale[0]).reshape(1, 32)
        if l != L - 1:
            h2 = _bn_mid(z, scale, shift).reshape(2 * NT, 16)
        else:
            out = _bn_final(z, scale, shift)
    return out
